# baseline (speedup 1.0000x reference)
import jax
import jax.numpy as jnp
from jax import lax
from jax.experimental import pallas as pl
from jax.experimental.pallas import tpu as pltpu
from jax.experimental.pallas import tpu_sc as plsc

N = 10000
D = 128
H = 64
E = 160000
ET = 16
SIB_ID = 3

NC = 2
NS = 16
NW = NC * NS
EPAD = 163840
CPT = EPAD // NW
KCH = 128
NCHUNK = CPT // KCH
NPAD = 10240
RPT = NPAD // NS
EB = 2048
NSB = EPAD // EB
RB = 1024
NRB = NPAD // RB


def _prep_body(c_ref, x_ref, w_ref, b_ref, w1a_ref, w1b_ref, b1_ref,
               xtan_ref, xlin_ref, pq_ref):
    c = jnp.clip(c_ref[0, 0], 0.1, 10.0)
    sq = jnp.sqrt(c)
    x = x_ref[...]
    nrm = jnp.sqrt(jnp.sum(x * x, axis=-1, keepdims=True))
    nrm = jnp.clip(nrm, 1e-7, None)
    a = sq * nrm
    xt = jnp.log(a + jnp.sqrt(a * a + 1.0)) / a * x
    xl = jnp.dot(xt, w_ref[...].T, preferred_element_type=jnp.float32) + b_ref[...]
    xtan_ref[...] = xt
    xlin_ref[...] = xl
    pp = jnp.dot(xl, w1a_ref[...].T,
                 preferred_element_type=jnp.float32) + b1_ref[...]
    qq = jnp.dot(xl, w1b_ref[...].T,
                 preferred_element_type=jnp.float32)
    pq_ref[...] = jnp.concatenate([pp, qq], axis=-1)


def _prep_call(c, x, w, b, w1a, w1b, b1):
    f32 = jnp.float32
    return pl.pallas_call(
        _prep_body,
        grid=(NRB,),
        in_specs=[
            pl.BlockSpec((1, 1), lambda i: (0, 0)),
            pl.BlockSpec((RB, D), lambda i: (i, 0)),
            pl.BlockSpec((D, D), lambda i: (0, 0)),
            pl.BlockSpec((1, D), lambda i: (0, 0)),
            pl.BlockSpec((H, D), lambda i: (0, 0)),
            pl.BlockSpec((H, D), lambda i: (0, 0)),
            pl.BlockSpec((1, H), lambda i: (0, 0)),
        ],
        out_specs=[
            pl.BlockSpec((RB, D), lambda i: (i, 0)),
            pl.BlockSpec((RB, D), lambda i: (i, 0)),
            pl.BlockSpec((RB, 2 * H), lambda i: (i, 0)),
        ],
        out_shape=[
            jax.ShapeDtypeStruct((NPAD, D), f32),
            jax.ShapeDtypeStruct((NPAD, D), f32),
            jax.ShapeDtypeStruct((NPAD, 2 * H), f32),
        ],
    )(c, x, w, b, w1a, w1b, b1)


def _rproj_body(etab_ref, w1c_ref, r_ref):
    r_ref[...] = jnp.dot(etab_ref[...], w1c_ref[...].T,
                         preferred_element_type=jnp.float32)


def _rproj_call(etab, w1c):
    return pl.pallas_call(
        _rproj_body,
        out_shape=jax.ShapeDtypeStruct((ET, H), jnp.float32),
    )(etab, w1c)


def _score_body(sib_ref, z_ref, etev_ref, etod_ref, ewev_ref, ewod_ref,
                r_ref, w2d_ref, sev_ref, sod_ref, bmax_ref):
    z = z_ref[...]
    etev = etev_ref[0, 0, :]
    etod = etod_ref[0, 0, :]
    iot = lax.broadcasted_iota(jnp.int32, (EB // 2, ET), 1)
    ohev = (etev[:, None] == iot).astype(jnp.float32)
    ohod = (etod[:, None] == iot).astype(jnp.float32)
    radd = jnp.concatenate(
        [jnp.dot(ohev, r_ref[...], preferred_element_type=jnp.float32),
         jnp.dot(ohod, r_ref[...], preferred_element_type=jnp.float32)],
        axis=-1)
    z = z + radd
    h = z / (1.0 + jnp.exp(-z))
    t = h * w2d_ref[...]
    sev = jnp.sum(t[:, :H], axis=1)
    sod = jnp.sum(t[:, H:], axis=1)

    def fix(s, ew, et):
        lew = jnp.where(ew < 0.0, -1e30,
                        jnp.log(jnp.clip(ew, 1e-6, None)))
        return s + lew + jnp.where(et == SIB_ID, sib_ref[0, 0], 0.0)

    sev = fix(sev, ewev_ref[0, 0, :], etev)
    sod = fix(sod, ewod_ref[0, 0, :], etod)
    sev_ref[0, 0, :] = sev
    sod_ref[0, 0, :] = sod
    m = jnp.maximum(jnp.max(sev), jnp.max(sod))
    i = pl.program_id(0)

    @pl.when(i == 0)
    def _():
        bmax_ref[0, 0, :] = jnp.full((16,), m, dtype=jnp.float32)

    @pl.when(i != 0)
    def _():
        bmax_ref[0, 0, :] = jnp.maximum(bmax_ref[0, 0, :], m)


def _score_call(sib, z2, etev, etod, ewev, ewod, r, w2d):
    f32 = jnp.float32
    eh = EB // 2
    return pl.pallas_call(
        _score_body,
        grid=(NSB,),
        in_specs=[
            pl.BlockSpec((1, 1), lambda i: (0, 0)),
            pl.BlockSpec((eh, 2 * H), lambda i: (i, 0)),
            pl.BlockSpec((1, 1, eh), lambda i: (i, 0, 0)),
            pl.BlockSpec((1, 1, eh), lambda i: (i, 0, 0)),
            pl.BlockSpec((1, 1, eh), lambda i: (i, 0, 0)),
            pl.BlockSpec((1, 1, eh), lambda i: (i, 0, 0)),
            pl.BlockSpec((ET, H), lambda i: (0, 0)),
            pl.BlockSpec((1, 2 * H), lambda i: (0, 0)),
        ],
        out_specs=[
            pl.BlockSpec((1, 1, eh), lambda i: (i, 0, 0)),
            pl.BlockSpec((1, 1, eh), lambda i: (i, 0, 0)),
            pl.BlockSpec((1, 1, 16), lambda i: (0, 0, 0)),
        ],
        out_shape=[
            jax.ShapeDtypeStruct((NSB, 1, eh), f32),
            jax.ShapeDtypeStruct((NSB, 1, eh), f32),
            jax.ShapeDtypeStruct((1, 1, 16), f32),
        ],
    )(sib, z2, etev, etod, ewev, ewod, r, w2d)


def _post_body(c_ref, xtan_ref, a0_ref, a1_ref, d0_ref, d1_ref,
               g_ref, beta_ref, out_ref):
    den = d0_ref[0] + d1_ref[0] + 1e-16
    out = xtan_ref[...] + (a0_ref[0] + a1_ref[0]) / den
    mean = jnp.mean(out, axis=-1, keepdims=True)
    var = jnp.mean((out - mean) * (out - mean), axis=-1, keepdims=True)
    out = (out - mean) * lax.rsqrt(var + 1e-5) * g_ref[...] + beta_ref[...]
    c = jnp.clip(c_ref[0, 0], 0.1, 10.0)
    sq = jnp.sqrt(c)
    nrm = jnp.sqrt(jnp.sum(out * out, axis=-1, keepdims=True))
    nrm = jnp.clip(nrm, 1e-7, None)
    a = sq * nrm
    sinh = 0.5 * (jnp.exp(a) - jnp.exp(-a))
    out_ref[...] = sinh / a * out


def _post_call(c, xtan, agg, den3, g, beta):
    return pl.pallas_call(
        _post_body,
        grid=(NRB,),
        in_specs=[
            pl.BlockSpec((1, 1), lambda i: (0, 0)),
            pl.BlockSpec((RB, D), lambda i: (i, 0)),
            pl.BlockSpec((1, RB, D), lambda i: (0, i, 0)),
            pl.BlockSpec((1, RB, D), lambda i: (1, i, 0)),
            pl.BlockSpec((1, RB, 1), lambda i: (0, i, 0)),
            pl.BlockSpec((1, RB, 1), lambda i: (1, i, 0)),
            pl.BlockSpec((1, D), lambda i: (0, 0)),
            pl.BlockSpec((1, D), lambda i: (0, 0)),
        ],
        out_specs=pl.BlockSpec((RB, D), lambda i: (i, 0)),
        out_shape=jax.ShapeDtypeStruct((NPAD, D), jnp.float32),
    )(c, xtan, agg, agg, den3, den3, g, beta)


def _sc_gather_body(pq_hbm, src2, dst2, z_out,
                    sidx, didx, dbuf, sbuf, dbuf1, sbuf1, zbuf, zbuf1,
                    dsem0, ssem0, dsem1, ssem1):
    cid = lax.axis_index("c")
    sid = lax.axis_index("s")
    wid = sid * NC + cid
    row0 = pl.multiple_of(wid * NCHUNK, 8)
    pltpu.sync_copy(src2.at[pl.ds(row0, NCHUNK)], sidx)
    pltpu.sync_copy(dst2.at[pl.ds(row0, NCHUNK)], didx)

    dbufs = (dbuf, dbuf1)
    sbufs = (sbuf, sbuf1)
    zbufs = (zbuf, zbuf1)
    dsems = (dsem0, dsem1)
    ssems = (ssem0, ssem1)

    def fire(j, b):
        pltpu.async_copy(pq_hbm.at[didx.at[j]], dbufs[b], dsems[b])
        pltpu.async_copy(pq_hbm.at[sidx.at[j]], sbufs[b], ssems[b])

    def waitb(j, b):
        pltpu.make_async_copy(pq_hbm.at[didx.at[j]], dbufs[b],
                              dsems[b]).wait()
        pltpu.make_async_copy(pq_hbm.at[sidx.at[j]], sbufs[b],
                              ssems[b]).wait()

    def compute_out(j, b):
        db = dbufs[b]
        sb = sbufs[b]
        zb = zbufs[b]

        def addrow(r, c2):
            e0 = r * 2
            e1 = e0 + 1
            for f in range(H // 16):
                sl = pl.ds(f * 16, 16)
                sh = pl.ds(H + f * 16, 16)
                zb[r, sl] = db[e0, sl] + sb[e0, sh]
                zb[r, sh] = db[e1, sl] + sb[e1, sh]
            return c2
        lax.fori_loop(0, KCH // 2, addrow, 0)

        off = pl.multiple_of((wid * CPT + j * KCH) // 2, 8)
        pltpu.sync_copy(zb, z_out.at[pl.ds(off, KCH // 2)])

    fire(0, 0)

    def body(g, carry):
        j0 = 2 * g
        fire(j0 + 1, 1)
        waitb(j0, 0)
        compute_out(j0, 0)

        @pl.when(g + 1 < NCHUNK // 2)
        def _():
            fire(j0 + 2, 0)

        waitb(j0 + 1, 1)
        compute_out(j0 + 1, 1)
        return carry

    lax.fori_loop(0, NCHUNK // 2, body, 0)


def _sc_gather_call(pq, src2, dst2):
    f32 = jnp.float32
    mesh = plsc.VectorSubcoreMesh(core_axis_name="c", subcore_axis_name="s")
    return pl.kernel(
        _sc_gather_body,
        out_type=jax.ShapeDtypeStruct((EPAD // 2, 2 * H), f32),
        mesh=mesh,
        scratch_types=[
            pltpu.VMEM((NCHUNK, KCH), jnp.int32),
            pltpu.VMEM((NCHUNK, KCH), jnp.int32),
            pltpu.VMEM((KCH, 2 * H), f32),
            pltpu.VMEM((KCH, 2 * H), f32),
            pltpu.VMEM((KCH, 2 * H), f32),
            pltpu.VMEM((KCH, 2 * H), f32),
            pltpu.VMEM((KCH // 2, 2 * H), f32),
            pltpu.VMEM((KCH // 2, 2 * H), f32),
            pltpu.SemaphoreType.DMA,
            pltpu.SemaphoreType.DMA,
            pltpu.SemaphoreType.DMA,
            pltpu.SemaphoreType.DMA,
        ],
    )(pq, src2, dst2)


GC = 8
NGRP = NCHUNK // GC


def _sc_agg_body(s_hbm, ew2, et2, src2, dst2, bmax_hbm, xlin_hbm, etab_hbm,
                 agg_out, den_out,
                 agg_sh, den_sh,
                 sidx, didx, sloc, ewloc, etloc, mbuf, etab_loc,
                 xbuf, xbuf1, exd, exd1, zden, sem, sem1,
                 asem0, asem1, nsem0, nsem1):
    cid = lax.axis_index("c")
    sid = lax.axis_index("s")
    wid = sid * NC + cid

    pltpu.sync_copy(bmax_hbm, mbuf)
    pltpu.sync_copy(etab_hbm, etab_loc)

    gmax = mbuf[...][0]

    def zrow(r, carry):
        for f in range(D // 16):
            xbuf[r, pl.ds(f * 16, 16)] = jnp.zeros((16,), jnp.float32)
        return carry
    lax.fori_loop(0, KCH, zrow, 0)

    def zden_step(i, carry):
        zden[pl.ds(i * 16, 16)] = jnp.zeros((16,), jnp.float32)
        return carry
    lax.fori_loop(0, RPT // 16, zden_step, 0)

    arow0 = pl.multiple_of(sid * RPT, 128)
    for k in range(RPT // KCH):
        pltpu.sync_copy(xbuf, agg_sh.at[pl.ds(arow0 + k * KCH, KCH)])
    pltpu.sync_copy(zden, den_sh.at[pl.ds(arow0, RPT)])
    plsc.subcore_barrier()

    xbufs = (xbuf, xbuf1)
    exds = (exd, exd1)
    sems = (sem, sem1)
    asems = (asem0, asem1)
    nsems = (nsem0, nsem1)

    def fire(jj, b):
        pltpu.async_copy(xlin_hbm.at[sidx.at[jj]], xbufs[b], sems[b])

    def waitb(jj, b):
        pltpu.make_async_copy(xlin_hbm.at[sidx.at[jj]], xbufs[b],
                              sems[b]).wait()

    def group(g, carry):
        row0 = pl.multiple_of(wid * NCHUNK + g * GC, 8)
        pltpu.sync_copy(src2.at[pl.ds(row0, GC)], sidx)
        pltpu.sync_copy(dst2.at[pl.ds(row0, GC)], didx)
        pltpu.sync_copy(ew2.at[pl.ds(row0, GC)], ewloc)
        pltpu.sync_copy(et2.at[pl.ds(row0, GC)], etloc)
        pltpu.sync_copy(
            s_hbm.at[pl.ds(pl.multiple_of(wid * CPT + g * GC * KCH, 128),
                           GC * KCH)], sloc)

        def process(jj, b):
            xb = xbufs[b]
            eb = exds[b]

            def egrp(i, c2):
                sv = sloc[pl.ds(jj * KCH + i * 16, 16)]
                e = jnp.exp(sv - gmax)
                eb[pl.ds(i * 16, 16)] = e
                wvec = e * ewloc[jj, pl.ds(i * 16, 16)]
                tvec = etloc[jj, pl.ds(i * 16, 16)]
                base = i * 16
                for q in range(16):
                    w = wvec[q]
                    t = tvec[q]
                    k = base + q
                    for f in range(D // 16):
                        sl = pl.ds(f * 16, 16)
                        xb[k, sl] = (xb[k, sl] + etab_loc[t, sl]) * w
                return c2
            lax.fori_loop(0, KCH // 16, egrp, 0)

            pltpu.async_copy(xb, agg_sh.at[didx.at[jj]], asems[b], add=True)
            pltpu.async_copy(eb, den_sh.at[didx.at[jj]], nsems[b], add=True)

        def wait_scat(jj, b):
            pltpu.make_async_copy(xbufs[b], agg_sh.at[didx.at[jj]],
                                  asems[b]).wait()
            pltpu.make_async_copy(exds[b], den_sh.at[didx.at[jj]],
                                  nsems[b]).wait()

        fire(0, 0)
        fire(1, 1)

        def pair(p, c1):
            j0 = 2 * p
            waitb(j0, 0)
            process(j0, 0)
            waitb(j0 + 1, 1)
            process(j0 + 1, 1)
            wait_scat(j0, 0)
            wait_scat(j0 + 1, 1)

            @pl.when(p + 1 < GC // 2)
            def _():
                fire(j0 + 2, 0)
                fire(j0 + 3, 1)
            return c1

        lax.fori_loop(0, GC // 2, pair, 0)
        return carry

    lax.fori_loop(0, NGRP, group, 0)
    plsc.subcore_barrier()

    pltpu.sync_copy(agg_sh.at[pl.ds(arow0, RPT)],
                    agg_out.at[cid, pl.ds(arow0, RPT)])
    pltpu.sync_copy(den_sh.at[pl.ds(arow0, RPT)],
                    den_out.at[cid, pl.ds(arow0, RPT)])


def _sc_agg_call(s, ew2, et2, src2, dst2, bmax, xlin, etab):
    f32 = jnp.float32
    i32 = jnp.int32
    mesh = plsc.VectorSubcoreMesh(core_axis_name="c", subcore_axis_name="s")
    return pl.kernel(
        _sc_agg_body,
        out_type=[
            jax.ShapeDtypeStruct((NC, NPAD, D), f32),
            jax.ShapeDtypeStruct((NC, NPAD), f32),
        ],
        mesh=mesh,
        scratch_types=[
            pltpu.VMEM_SHARED((NPAD, D), f32),
            pltpu.VMEM_SHARED((NPAD,), f32),
            pltpu.VMEM((GC, KCH), i32),
            pltpu.VMEM((GC, KCH), i32),
            pltpu.VMEM((GC * KCH,), f32),
            pltpu.VMEM((GC, KCH), f32),
            pltpu.VMEM((GC, KCH), i32),
            pltpu.VMEM((16,), f32),
            pltpu.VMEM((ET, D), f32),
            pltpu.VMEM((KCH, D), f32),
            pltpu.VMEM((KCH, D), f32),
            pltpu.VMEM((KCH,), f32),
            pltpu.VMEM((KCH,), f32),
            pltpu.VMEM((RPT,), f32),
            pltpu.SemaphoreType.DMA,
            pltpu.SemaphoreType.DMA,
            pltpu.SemaphoreType.DMA,
            pltpu.SemaphoreType.DMA,
            pltpu.SemaphoreType.DMA,
            pltpu.SemaphoreType.DMA,
        ],
    )(s, ew2, et2, src2, dst2, bmax, xlin, etab)


def kernel(x_hyp, edge_index, edge_types, edge_weights, lin_W, lin_b,
           ln_g, ln_b, curvature, edge_emb, attn_W1, attn_b1, attn_W2,
           attn_b2, sibling_boost):
    f32 = jnp.float32
    i32 = jnp.int32
    src = edge_index[0]
    dst = edge_index[1]
    padn = EPAD - E
    spread = jnp.arange(padn, dtype=i32) % N
    src_p = jnp.concatenate([src, spread])
    dst_p = jnp.concatenate([dst, spread])
    et_p = jnp.concatenate([edge_types, jnp.zeros((padn,), i32)])
    ew_p = jnp.concatenate([edge_weights, jnp.full((padn,), -1.0, f32)])
    src2 = src_p.reshape(EPAD // KCH, KCH)
    dst2 = dst_p.reshape(EPAD // KCH, KCH)
    et2 = et_p.reshape(EPAD // KCH, KCH)
    ew2 = ew_p.reshape(EPAD // KCH, KCH)
    eh = EB // 2
    etev = et_p[0::2].reshape(NSB, 1, eh)
    etod = et_p[1::2].reshape(NSB, 1, eh)
    ewev = ew_p[0::2].reshape(NSB, 1, eh)
    ewod = ew_p[1::2].reshape(NSB, 1, eh)

    x = jnp.pad(x_hyp, ((0, NPAD - N), (0, 0)))
    for l in range(lin_W.shape[0]):
        c = curvature[l].reshape(1, 1)
        w = lin_W[l]
        b = lin_b[l].reshape(1, D)
        w1 = attn_W1[l]
        w1a = w1[:, :D]
        w1b = w1[:, D:2 * D]
        w1c = w1[:, 2 * D:]
        b1 = attn_b1[l].reshape(1, H)
        w2 = attn_W2[l]
        sib = sibling_boost[l].reshape(1, 1)
        etab = edge_emb[l]
        g = ln_g[l].reshape(1, D)
        beta = ln_b[l].reshape(1, D)

        xtan, xlin, pq = _prep_call(c, x, w, b, w1a, w1b, b1)
        r = _rproj_call(etab, w1c)
        z2 = _sc_gather_call(pq, src2, dst2)
        w2d = jnp.concatenate([w2, w2], axis=-1)
        sev, sod, bmax3 = _score_call(sib, z2, etev, etod, ewev, ewod, r, w2d)
        s = jnp.stack([sev.reshape(EPAD // 2), sod.reshape(EPAD // 2)],
                      axis=-1).reshape(EPAD)
        bmax = bmax3.reshape(16)
        agg, den = _sc_agg_call(s, ew2, et2, src2, dst2, bmax, xlin, etab)
        den3 = den.reshape(NC, NPAD, 1)
        x = _post_call(c, xtan, agg, den3, g, beta)
    return x[:N]


# speedup vs baseline: 1.2856x; 1.2856x over previous
import jax
import jax.numpy as jnp
from jax import lax
from jax.experimental import pallas as pl
from jax.experimental.pallas import tpu as pltpu
from jax.experimental.pallas import tpu_sc as plsc

N = 10000
D = 128
H = 64
E = 160000
ET = 16
SIB_ID = 3

NC = 2
NS = 16
NW = NC * NS
EPAD = 163840
CPT = EPAD // NW
KCH = 128
NCHUNK = CPT // KCH
NPAD = 10240
RPT = NPAD // NS
EB = 2048
NSB = EPAD // EB
RB = 1024
NRB = NPAD // RB


def _prep_body(c_ref, x_ref, w_ref, b_ref, w1a_ref, w1b_ref, b1_ref,
               xtan_ref, xlin_ref, pq_ref):
    c = jnp.clip(c_ref[0, 0], 0.1, 10.0)
    sq = jnp.sqrt(c)
    x = x_ref[...]
    nrm = jnp.sqrt(jnp.sum(x * x, axis=-1, keepdims=True))
    nrm = jnp.clip(nrm, 1e-7, None)
    a = sq * nrm
    xt = jnp.log(a + jnp.sqrt(a * a + 1.0)) / a * x
    xl = jnp.dot(xt, w_ref[...].T, preferred_element_type=jnp.float32) + b_ref[...]
    xtan_ref[...] = xt
    xlin_ref[...] = xl
    pp = jnp.dot(xl, w1a_ref[...].T,
                 preferred_element_type=jnp.float32) + b1_ref[...]
    qq = jnp.dot(xl, w1b_ref[...].T,
                 preferred_element_type=jnp.float32)
    pq_ref[...] = jnp.concatenate([pp, qq], axis=-1)


def _prep_call(c, x, w, b, w1a, w1b, b1):
    f32 = jnp.float32
    return pl.pallas_call(
        _prep_body,
        grid=(NRB,),
        in_specs=[
            pl.BlockSpec((1, 1), lambda i: (0, 0)),
            pl.BlockSpec((RB, D), lambda i: (i, 0)),
            pl.BlockSpec((D, D), lambda i: (0, 0)),
            pl.BlockSpec((1, D), lambda i: (0, 0)),
            pl.BlockSpec((H, D), lambda i: (0, 0)),
            pl.BlockSpec((H, D), lambda i: (0, 0)),
            pl.BlockSpec((1, H), lambda i: (0, 0)),
        ],
        out_specs=[
            pl.BlockSpec((RB, D), lambda i: (i, 0)),
            pl.BlockSpec((RB, D), lambda i: (i, 0)),
            pl.BlockSpec((RB, 2 * H), lambda i: (i, 0)),
        ],
        out_shape=[
            jax.ShapeDtypeStruct((NPAD, D), f32),
            jax.ShapeDtypeStruct((NPAD, D), f32),
            jax.ShapeDtypeStruct((NPAD, 2 * H), f32),
        ],
    )(c, x, w, b, w1a, w1b, b1)


def _rproj_body(etab_ref, w1c_ref, r_ref):
    r_ref[...] = jnp.dot(etab_ref[...], w1c_ref[...].T,
                         preferred_element_type=jnp.float32)


def _rproj_call(etab, w1c):
    return pl.pallas_call(
        _rproj_body,
        out_shape=jax.ShapeDtypeStruct((ET, H), jnp.float32),
    )(etab, w1c)


def _score_body(sib_ref, z_ref, etev_ref, etod_ref, ewev_ref, ewod_ref,
                r_ref, w2d_ref, sev_ref, sod_ref, bmax_ref):
    z = z_ref[...]
    etev = etev_ref[0, 0, :]
    etod = etod_ref[0, 0, :]
    iot = lax.broadcasted_iota(jnp.int32, (EB // 2, ET), 1)
    ohev = (etev[:, None] == iot).astype(jnp.float32)
    ohod = (etod[:, None] == iot).astype(jnp.float32)
    radd = jnp.concatenate(
        [jnp.dot(ohev, r_ref[...], preferred_element_type=jnp.float32),
         jnp.dot(ohod, r_ref[...], preferred_element_type=jnp.float32)],
        axis=-1)
    z = z + radd
    h = z / (1.0 + jnp.exp(-z))
    t = h * w2d_ref[...]
    sev = jnp.sum(t[:, :H], axis=1)
    sod = jnp.sum(t[:, H:], axis=1)

    def fix(s, ew, et):
        lew = jnp.where(ew < 0.0, -1e30,
                        jnp.log(jnp.clip(ew, 1e-6, None)))
        return s + lew + jnp.where(et == SIB_ID, sib_ref[0, 0], 0.0)

    sev = fix(sev, ewev_ref[0, 0, :], etev)
    sod = fix(sod, ewod_ref[0, 0, :], etod)
    sev_ref[0, 0, :] = sev
    sod_ref[0, 0, :] = sod
    m = jnp.maximum(jnp.max(sev), jnp.max(sod))
    i = pl.program_id(0)

    @pl.when(i == 0)
    def _():
        bmax_ref[0, 0, :] = jnp.full((16,), m, dtype=jnp.float32)

    @pl.when(i != 0)
    def _():
        bmax_ref[0, 0, :] = jnp.maximum(bmax_ref[0, 0, :], m)


def _score_call(sib, z2, etev, etod, ewev, ewod, r, w2d):
    f32 = jnp.float32
    eh = EB // 2
    return pl.pallas_call(
        _score_body,
        grid=(NSB,),
        in_specs=[
            pl.BlockSpec((1, 1), lambda i: (0, 0)),
            pl.BlockSpec((eh, 2 * H), lambda i: (i, 0)),
            pl.BlockSpec((1, 1, eh), lambda i: (i, 0, 0)),
            pl.BlockSpec((1, 1, eh), lambda i: (i, 0, 0)),
            pl.BlockSpec((1, 1, eh), lambda i: (i, 0, 0)),
            pl.BlockSpec((1, 1, eh), lambda i: (i, 0, 0)),
            pl.BlockSpec((ET, H), lambda i: (0, 0)),
            pl.BlockSpec((1, 2 * H), lambda i: (0, 0)),
        ],
        out_specs=[
            pl.BlockSpec((1, 1, eh), lambda i: (i, 0, 0)),
            pl.BlockSpec((1, 1, eh), lambda i: (i, 0, 0)),
            pl.BlockSpec((1, 1, 16), lambda i: (0, 0, 0)),
        ],
        out_shape=[
            jax.ShapeDtypeStruct((NSB, 1, eh), f32),
            jax.ShapeDtypeStruct((NSB, 1, eh), f32),
            jax.ShapeDtypeStruct((1, 1, 16), f32),
        ],
    )(sib, z2, etev, etod, ewev, ewod, r, w2d)


def _post_body(c_ref, xtan_ref, a0_ref, a1_ref, d0_ref, d1_ref,
               c0_ref, c1_ref, etab_ref, g_ref, beta_ref, out_ref):
    den = d0_ref[0] + d1_ref[0] + 1e-16
    num = (a0_ref[0] + a1_ref[0]
           + jnp.dot(c0_ref[0] + c1_ref[0], etab_ref[...],
                     preferred_element_type=jnp.float32))
    out = xtan_ref[...] + num / den
    mean = jnp.mean(out, axis=-1, keepdims=True)
    var = jnp.mean((out - mean) * (out - mean), axis=-1, keepdims=True)
    out = (out - mean) * lax.rsqrt(var + 1e-5) * g_ref[...] + beta_ref[...]
    c = jnp.clip(c_ref[0, 0], 0.1, 10.0)
    sq = jnp.sqrt(c)
    nrm = jnp.sqrt(jnp.sum(out * out, axis=-1, keepdims=True))
    nrm = jnp.clip(nrm, 1e-7, None)
    a = sq * nrm
    sinh = 0.5 * (jnp.exp(a) - jnp.exp(-a))
    out_ref[...] = sinh / a * out


def _post_call(c, xtan, agg, den3, cnt3, etab, g, beta):
    return pl.pallas_call(
        _post_body,
        grid=(NRB,),
        in_specs=[
            pl.BlockSpec((1, 1), lambda i: (0, 0)),
            pl.BlockSpec((RB, D), lambda i: (i, 0)),
            pl.BlockSpec((1, RB, D), lambda i: (0, i, 0)),
            pl.BlockSpec((1, RB, D), lambda i: (1, i, 0)),
            pl.BlockSpec((1, RB, 1), lambda i: (0, i, 0)),
            pl.BlockSpec((1, RB, 1), lambda i: (1, i, 0)),
            pl.BlockSpec((1, RB, ET), lambda i: (0, i, 0)),
            pl.BlockSpec((1, RB, ET), lambda i: (1, i, 0)),
            pl.BlockSpec((ET, D), lambda i: (0, 0)),
            pl.BlockSpec((1, D), lambda i: (0, 0)),
            pl.BlockSpec((1, D), lambda i: (0, 0)),
        ],
        out_specs=pl.BlockSpec((RB, D), lambda i: (i, 0)),
        out_shape=jax.ShapeDtypeStruct((NPAD, D), jnp.float32),
    )(c, xtan, agg, agg, den3, den3, cnt3, cnt3, etab, g, beta)


def _sc_gather_body(pq_hbm, src2, dst2, z_out,
                    sidx, didx, dbuf, sbuf, dbuf1, sbuf1, zbuf, zbuf1,
                    dsem0, ssem0, dsem1, ssem1):
    cid = lax.axis_index("c")
    sid = lax.axis_index("s")
    wid = sid * NC + cid
    row0 = pl.multiple_of(wid * NCHUNK, 8)
    pltpu.sync_copy(src2.at[pl.ds(row0, NCHUNK)], sidx)
    pltpu.sync_copy(dst2.at[pl.ds(row0, NCHUNK)], didx)

    dbufs = (dbuf, dbuf1)
    sbufs = (sbuf, sbuf1)
    zbufs = (zbuf, zbuf1)
    dsems = (dsem0, dsem1)
    ssems = (ssem0, ssem1)

    def fire(j, b):
        pltpu.async_copy(pq_hbm.at[didx.at[j]], dbufs[b], dsems[b])
        pltpu.async_copy(pq_hbm.at[sidx.at[j]], sbufs[b], ssems[b])

    def waitb(j, b):
        pltpu.make_async_copy(pq_hbm.at[didx.at[j]], dbufs[b],
                              dsems[b]).wait()
        pltpu.make_async_copy(pq_hbm.at[sidx.at[j]], sbufs[b],
                              ssems[b]).wait()

    def compute_out(j, b):
        db = dbufs[b]
        sb = sbufs[b]
        zb = zbufs[b]

        for r in range(KCH // 2):
            e0 = r * 2
            e1 = e0 + 1
            for f in range(H // 16):
                sl = pl.ds(f * 16, 16)
                sh = pl.ds(H + f * 16, 16)
                zb[r, sl] = db[e0, sl] + sb[e0, sh]
                zb[r, sh] = db[e1, sl] + sb[e1, sh]

        off = pl.multiple_of((wid * CPT + j * KCH) // 2, 8)
        pltpu.sync_copy(zb, z_out.at[pl.ds(off, KCH // 2)])

    fire(0, 0)

    def body(g, carry):
        j0 = 2 * g
        fire(j0 + 1, 1)
        waitb(j0, 0)
        compute_out(j0, 0)

        @pl.when(g + 1 < NCHUNK // 2)
        def _():
            fire(j0 + 2, 0)

        waitb(j0 + 1, 1)
        compute_out(j0 + 1, 1)
        return carry

    lax.fori_loop(0, NCHUNK // 2, body, 0)


def _sc_gather_call(pq, src2, dst2):
    f32 = jnp.float32
    mesh = plsc.VectorSubcoreMesh(core_axis_name="c", subcore_axis_name="s")
    return pl.kernel(
        _sc_gather_body,
        out_type=jax.ShapeDtypeStruct((EPAD // 2, 2 * H), f32),
        mesh=mesh,
        scratch_types=[
            pltpu.VMEM((NCHUNK, KCH), jnp.int32),
            pltpu.VMEM((NCHUNK, KCH), jnp.int32),
            pltpu.VMEM((KCH, 2 * H), f32),
            pltpu.VMEM((KCH, 2 * H), f32),
            pltpu.VMEM((KCH, 2 * H), f32),
            pltpu.VMEM((KCH, 2 * H), f32),
            pltpu.VMEM((KCH // 2, 2 * H), f32),
            pltpu.VMEM((KCH // 2, 2 * H), f32),
            pltpu.SemaphoreType.DMA,
            pltpu.SemaphoreType.DMA,
            pltpu.SemaphoreType.DMA,
            pltpu.SemaphoreType.DMA,
        ],
    )(pq, src2, dst2)


GC = 8
NGRP = NCHUNK // GC


def _sc_agg_body(s_hbm, ew2, et2, src2, dst2, bmax_hbm, xlin_hbm,
                 agg_out, den_out, cnt_out,
                 agg_sh, den_sh, cnt_sh,
                 sidx, didx, sloc, ewloc, etloc, mbuf,
                 xbuf, exd, exw, cdx, zden, sem, asem, nsem, csem):
    cid = lax.axis_index("c")
    sid = lax.axis_index("s")
    wid = sid * NC + cid

    pltpu.sync_copy(bmax_hbm, mbuf)

    gmax = mbuf[...][0]

    for r in range(KCH):
        for f in range(D // 16):
            xbuf[r, pl.ds(f * 16, 16)] = jnp.zeros((16,), jnp.float32)
    for i in range(RPT // 16):
        zden[pl.ds(i * 16, 16)] = jnp.zeros((16,), jnp.float32)

    arow0 = pl.multiple_of(sid * RPT, 128)
    for k in range(RPT // KCH):
        pltpu.sync_copy(xbuf, agg_sh.at[pl.ds(arow0 + k * KCH, KCH)])
    pltpu.sync_copy(zden, den_sh.at[pl.ds(arow0, RPT)])
    crow0 = pl.multiple_of(sid * RPT * ET, 128)
    for k in range(ET):
        pltpu.sync_copy(zden, cnt_sh.at[pl.ds(crow0 + k * RPT, RPT)])
    plsc.subcore_barrier()

    def fire(jj):
        pltpu.async_copy(xlin_hbm.at[sidx.at[jj]], xbuf, sem)

    def waitb(jj):
        pltpu.make_async_copy(xlin_hbm.at[sidx.at[jj]], xbuf, sem).wait()

    def group(g, carry):
        row0 = pl.multiple_of(wid * NCHUNK + g * GC, 8)
        pltpu.sync_copy(src2.at[pl.ds(row0, GC)], sidx)
        pltpu.sync_copy(dst2.at[pl.ds(row0, GC)], didx)
        pltpu.sync_copy(ew2.at[pl.ds(row0, GC)], ewloc)
        pltpu.sync_copy(et2.at[pl.ds(row0, GC)], etloc)
        pltpu.sync_copy(
            s_hbm.at[pl.ds(pl.multiple_of(wid * CPT + g * GC * KCH, 128),
                           GC * KCH)], sloc)

        def chunk(jj, c1):
            waitb(jj)
            for i in range(KCH // 16):
                sv = sloc[pl.ds(jj * KCH + i * 16, 16)]
                e = jnp.exp(sv - gmax)
                exd[pl.ds(i * 16, 16)] = e
                wvec = e * ewloc[jj, pl.ds(i * 16, 16)]
                exw[pl.ds(i * 16, 16)] = wvec
                cdx[pl.ds(i * 16, 16)] = (
                    didx[jj, pl.ds(i * 16, 16)] * ET
                    + etloc[jj, pl.ds(i * 16, 16)])
                for q in range(16):
                    w = wvec[q]
                    k = i * 16 + q
                    for f in range(D // 16):
                        sl = pl.ds(f * 16, 16)
                        xbuf[k, sl] = xbuf[k, sl] * w

            pltpu.async_copy(xbuf, agg_sh.at[didx.at[jj]], asem, add=True)
            pltpu.async_copy(exd, den_sh.at[didx.at[jj]], nsem, add=True)
            pltpu.async_copy(exw, cnt_sh.at[cdx], csem, add=True)
            pltpu.make_async_copy(xbuf, agg_sh.at[didx.at[jj]], asem).wait()
            pltpu.make_async_copy(exd, den_sh.at[didx.at[jj]], nsem).wait()
            pltpu.make_async_copy(exw, cnt_sh.at[cdx], csem).wait()

            @pl.when(jj + 1 < GC)
            def _():
                fire(jj + 1)
            return c1

        fire(0)
        lax.fori_loop(0, GC, chunk, 0)
        return carry

    lax.fori_loop(0, NGRP, group, 0)
    plsc.subcore_barrier()

    pltpu.sync_copy(agg_sh.at[pl.ds(arow0, RPT)],
                    agg_out.at[cid, pl.ds(arow0, RPT)])
    pltpu.sync_copy(den_sh.at[pl.ds(arow0, RPT)],
                    den_out.at[cid, pl.ds(arow0, RPT)])
    pltpu.sync_copy(cnt_sh.at[pl.ds(crow0, RPT * ET)],
                    cnt_out.at[cid, pl.ds(crow0, RPT * ET)])


def _sc_agg_call(s, ew2, et2, src2, dst2, bmax, xlin):
    f32 = jnp.float32
    i32 = jnp.int32
    mesh = plsc.VectorSubcoreMesh(core_axis_name="c", subcore_axis_name="s")
    return pl.kernel(
        _sc_agg_body,
        out_type=[
            jax.ShapeDtypeStruct((NC, NPAD, D), f32),
            jax.ShapeDtypeStruct((NC, NPAD), f32),
            jax.ShapeDtypeStruct((NC, NPAD * ET), f32),
        ],
        mesh=mesh,
        scratch_types=[
            pltpu.VMEM_SHARED((NPAD, D), f32),
            pltpu.VMEM_SHARED((NPAD,), f32),
            pltpu.VMEM_SHARED((NPAD * ET,), f32),
            pltpu.VMEM((GC, KCH), i32),
            pltpu.VMEM((GC, KCH), i32),
            pltpu.VMEM((GC * KCH,), f32),
            pltpu.VMEM((GC, KCH), f32),
            pltpu.VMEM((GC, KCH), i32),
            pltpu.VMEM((16,), f32),
            pltpu.VMEM((KCH, D), f32),
            pltpu.VMEM((KCH,), f32),
            pltpu.VMEM((KCH,), f32),
            pltpu.VMEM((KCH,), i32),
            pltpu.VMEM((RPT,), f32),
            pltpu.SemaphoreType.DMA,
            pltpu.SemaphoreType.DMA,
            pltpu.SemaphoreType.DMA,
            pltpu.SemaphoreType.DMA,
        ],
    )(s, ew2, et2, src2, dst2, bmax, xlin)


def kernel(x_hyp, edge_index, edge_types, edge_weights, lin_W, lin_b,
           ln_g, ln_b, curvature, edge_emb, attn_W1, attn_b1, attn_W2,
           attn_b2, sibling_boost):
    f32 = jnp.float32
    i32 = jnp.int32
    src = edge_index[0]
    dst = edge_index[1]
    padn = EPAD - E
    spread = jnp.arange(padn, dtype=i32) % N
    src_p = jnp.concatenate([src, spread])
    dst_p = jnp.concatenate([dst, spread])
    et_p = jnp.concatenate([edge_types, jnp.zeros((padn,), i32)])
    ew_p = jnp.concatenate([edge_weights, jnp.full((padn,), -1.0, f32)])
    src2 = src_p.reshape(EPAD // KCH, KCH)
    dst2 = dst_p.reshape(EPAD // KCH, KCH)
    et2 = et_p.reshape(EPAD // KCH, KCH)
    ew2 = ew_p.reshape(EPAD // KCH, KCH)
    eh = EB // 2
    etev = et_p[0::2].reshape(NSB, 1, eh)
    etod = et_p[1::2].reshape(NSB, 1, eh)
    ewev = ew_p[0::2].reshape(NSB, 1, eh)
    ewod = ew_p[1::2].reshape(NSB, 1, eh)

    x = jnp.pad(x_hyp, ((0, NPAD - N), (0, 0)))
    for l in range(lin_W.shape[0]):
        c = curvature[l].reshape(1, 1)
        w = lin_W[l]
        b = lin_b[l].reshape(1, D)
        w1 = attn_W1[l]
        w1a = w1[:, :D]
        w1b = w1[:, D:2 * D]
        w1c = w1[:, 2 * D:]
        b1 = attn_b1[l].reshape(1, H)
        w2 = attn_W2[l]
        sib = sibling_boost[l].reshape(1, 1)
        etab = edge_emb[l]
        g = ln_g[l].reshape(1, D)
        beta = ln_b[l].reshape(1, D)

        xtan, xlin, pq = _prep_call(c, x, w, b, w1a, w1b, b1)
        r = _rproj_call(etab, w1c)
        z2 = _sc_gather_call(pq, src2, dst2)
        w2d = jnp.concatenate([w2, w2], axis=-1)
        sev, sod, bmax3 = _score_call(sib, z2, etev, etod, ewev, ewod, r, w2d)
        s = jnp.stack([sev.reshape(EPAD // 2), sod.reshape(EPAD // 2)],
                      axis=-1).reshape(EPAD)
        bmax = bmax3.reshape(16)
        agg, den, cnt = _sc_agg_call(s, ew2, et2, src2, dst2, bmax, xlin)
        den3 = den.reshape(NC, NPAD, 1)
        cnt3 = cnt.reshape(NC, NPAD, ET)
        x = _post_call(c, xtan, agg, den3, cnt3, etab, g, beta)
    return x[:N]
